# baseline (speedup 1.0000x reference)
import functools

import jax
import jax.numpy as jnp
from jax import lax
from jax.experimental import pallas as pl
from jax.experimental.pallas import tpu as pltpu
from jax.experimental.pallas import tpu_sc as plsc

_B = 16384
_D = 32
_NC = 2
_NS = 16
_NW = _NC * _NS
_BPW = _B // _NW
_IC = 128
_NCHUNK = _BPW // _IC
_L = 16


def _sc_body(heads_hbm, tails_hbm, rels_hbm, ent_hbm, rel_hbm, out_hbm,
             hidx, tidx, ridx, e1, e2, r, out_v, sem):
    wid = lax.axis_index("s") * _NC + lax.axis_index("c")
    row0 = wid * _NCHUNK
    pltpu.sync_copy(heads_hbm.at[pl.ds(row0, _NCHUNK)], hidx)
    pltpu.sync_copy(tails_hbm.at[pl.ds(row0, _NCHUNK)], tidx)
    pltpu.sync_copy(rels_hbm.at[pl.ds(row0, _NCHUNK)], ridx)

    copies = []
    for j in range(_NCHUNK):
        dst = pl.ds(j * _IC, _IC)
        copies.append(pltpu.async_copy(ent_hbm.at[hidx.at[j]], e1.at[dst], sem))
        copies.append(pltpu.async_copy(ent_hbm.at[tidx.at[j]], e2.at[dst], sem))
        copies.append(pltpu.async_copy(rel_hbm.at[ridx.at[j]], r.at[dst], sem))
    for c in copies:
        c.wait()

    iota = lax.iota(jnp.int32, _L)

    def chunk(ci, carry):
        rows = pl.multiple_of(ci * _L, _L) + iota
        acc = jnp.zeros((_L,), jnp.float32)
        for d in range(_D):
            col = jnp.full((_L,), d, jnp.int32)
            a = plsc.load_gather(e1, [rows, col])
            b = plsc.load_gather(r, [rows, col])
            c = plsc.load_gather(e2, [rows, col])
            acc = acc + a * b * c
        pred = 1.0 / (1.0 + jnp.exp(-acc))
        out_v[pl.ds(pl.multiple_of(ci * _L, _L), _L)] = pred
        return carry

    lax.fori_loop(0, _BPW // _L, chunk, 0)
    pltpu.sync_copy(out_v, out_hbm.at[pl.ds(wid * _BPW, _BPW)])


_sc_call = functools.partial(
    pl.kernel,
    out_type=jax.ShapeDtypeStruct((_B,), jnp.float32),
    mesh=plsc.VectorSubcoreMesh(core_axis_name="c", subcore_axis_name="s"),
    compiler_params=pltpu.CompilerParams(
        use_tc_tiling_on_sc=False, needs_layout_passes=False
    ),
    scratch_types=[
        pltpu.VMEM((_NCHUNK, _IC), jnp.int32),
        pltpu.VMEM((_NCHUNK, _IC), jnp.int32),
        pltpu.VMEM((_NCHUNK, _IC), jnp.int32),
        pltpu.VMEM((_BPW, _D), jnp.float32),
        pltpu.VMEM((_BPW, _D), jnp.float32),
        pltpu.VMEM((_BPW, _D), jnp.float32),
        pltpu.VMEM((_BPW,), jnp.float32),
        pltpu.SemaphoreType.DMA,
    ],
)(_sc_body)


@jax.jit
def kernel(heads, tails, relations, entity_embedding, relation_embedding):
    h2 = heads.reshape(_B // _IC, _IC)
    t2 = tails.reshape(_B // _IC, _IC)
    r2 = relations.reshape(_B // _IC, _IC)
    return _sc_call(h2, t2, r2, entity_embedding, relation_embedding)
